# baseline (speedup 1.0000x reference)
import jax
import jax.numpy as jnp
from jax.experimental import pallas as pl
from jax.experimental.pallas import tpu as pltpu

B, M, N, K = 4, 2048, 8192, 32
C_IN, C_MID, C_OUT = 64, 64, 128


def _mlp_body(x_ref, w0_ref, b0_ref, w1_ref, b1_ref, out_ref):
    x = x_ref[0]
    h = jnp.maximum(
        jax.lax.dot_general(w0_ref[...], x, (((1,), (0,)), ((), ())),
                            preferred_element_type=jnp.float32) + b0_ref[...], 0.0)
    h = jnp.maximum(
        jax.lax.dot_general(w1_ref[...], h, (((1,), (0,)), ((), ())),
                            preferred_element_type=jnp.float32) + b1_ref[...], 0.0)
    out_ref[0] = h


def kernel(query_xyz, support_xyz, features, query_idx, W0, b0, W1, b1):
    x = jnp.concatenate([features, jnp.transpose(support_xyz, (0, 2, 1))], axis=1)
    TN = 2048
    h = pl.pallas_call(
        _mlp_body,
        grid=(B, N // TN),
        in_specs=[
            pl.BlockSpec((1, C_IN + 3, TN), lambda b, n: (b, 0, n)),
            pl.BlockSpec((C_MID, C_IN + 3), lambda b, n: (0, 0)),
            pl.BlockSpec((C_MID, 1), lambda b, n: (0, 0)),
            pl.BlockSpec((C_OUT, C_MID), lambda b, n: (0, 0)),
            pl.BlockSpec((C_OUT, 1), lambda b, n: (0, 0)),
        ],
        out_specs=pl.BlockSpec((1, C_OUT, TN), lambda b, n: (b, 0, n)),
        out_shape=jax.ShapeDtypeStruct((B, C_OUT, N), jnp.float32),
    )(x, W0, b0[:, None], W1, b1[:, None])

    q2 = jnp.sum(query_xyz * query_xyz, axis=-1, keepdims=True)
    s2 = jnp.sum(support_xyz * support_xyz, axis=-1)[:, None, :]
    d2 = q2 - 2.0 * jnp.einsum('bmc,bnc->bmn', query_xyz, support_xyz) + s2
    _, idx = jax.lax.top_k(-d2, K)
    fj = jnp.take_along_axis(h[:, :, None, :], idx[:, None, :, :], axis=-1)
    return jnp.max(fj, axis=-1)


# speedup vs baseline: 8.6139x; 8.6139x over previous
import functools

import jax
import jax.numpy as jnp
from jax import lax
from jax.experimental import pallas as pl
from jax.experimental.pallas import tpu as pltpu
from jax.experimental.pallas import tpu_sc as plsc

B, M, N, K = 4, 2048, 8192, 32
C_IN, C_MID, C_OUT = 64, 64, 128
TM = 256
TN_MLP = 2048
NQ = B * M

NC, NS = 2, 16
NW = NC * NS
QS = NQ // NW
G = 4
NG = QS // G


def _mlp_body(x_ref, w0_ref, b0_ref, w1_ref, b1_ref, out_ref):
    x = x_ref[...]
    h = jnp.maximum(
        jnp.dot(x, w0_ref[...], preferred_element_type=jnp.float32)
        + b0_ref[...], 0.0)
    out_ref[...] = jnp.maximum(
        jnp.dot(h, w1_ref[...], preferred_element_type=jnp.float32)
        + b1_ref[...], 0.0)


def _knn_body(q_ref, sT_ref, idx_ref):
    b = pl.program_id(0)
    q = q_ref[0]
    sT = sT_ref[0]
    qx, qy, qz = q[:, 0:1], q[:, 1:2], q[:, 2:3]
    sx, sy, sz = sT[0:1, :], sT[1:2, :], sT[2:3, :]
    dot = jnp.dot(q, sT, preferred_element_type=jnp.float32)
    q2 = qx * qx + qy * qy + qz * qz
    s2 = sx * sx + sy * sy + sz * sz
    d2 = (q2 - 2.0 * dot) + s2
    iota = lax.broadcasted_iota(jnp.int32, (TM, N), 1)
    inf = jnp.float32(jnp.inf)
    cols = []
    for _ in range(K):
        rowmin = jnp.min(d2, axis=1, keepdims=True)
        eq = d2 == rowmin
        rowidx = jnp.min(jnp.where(eq, iota, N), axis=1, keepdims=True)
        cols.append(rowidx)
        d2 = jnp.where(iota == rowidx, inf, d2)
    idx_ref[0] = jnp.concatenate(cols, axis=1) + b * N


def _agg_body(hT_hbm, idx_hbm, out_hbm, idx_v, rows_v, out_v, sem):
    wid = lax.axis_index("s") * NC + lax.axis_index("c")
    qbase = wid * QS
    pltpu.sync_copy(idx_hbm.at[pl.ds(qbase * K, QS * K)], idx_v)

    def group(g, carry):
        pltpu.async_copy(hT_hbm.at[idx_v.at[pl.ds(g * (G * K), G * K)]],
                         rows_v, sem).wait()
        for qi in range(G):
            for cb in range(C_OUT // 16):
                def red(r, acc):
                    return jnp.maximum(acc, rows_v[qi * K + r,
                                                   pl.ds(cb * 16, 16)])
                acc = lax.fori_loop(1, K, red,
                                    rows_v[qi * K, pl.ds(cb * 16, 16)])
                out_v[g * G + qi, pl.ds(cb * 16, 16)] = acc
        return carry

    lax.fori_loop(0, NG, group, 0)
    pltpu.sync_copy(out_v, out_hbm.at[pl.ds(qbase, QS)])


@functools.cache
def _make_agg():
    return pl.kernel(
        _agg_body,
        mesh=plsc.VectorSubcoreMesh(core_axis_name="c", subcore_axis_name="s"),
        out_type=jax.ShapeDtypeStruct((NQ, C_OUT), jnp.float32),
        scratch_types=[
            pltpu.VMEM((QS * K,), jnp.int32),
            pltpu.VMEM((G * K, C_OUT), jnp.float32),
            pltpu.VMEM((QS, C_OUT), jnp.float32),
            pltpu.SemaphoreType.DMA,
        ],
    )


def kernel(query_xyz, support_xyz, features, query_idx, W0, b0, W1, b1):
    xT = jnp.concatenate(
        [jnp.transpose(features, (0, 2, 1)), support_xyz], axis=2)
    x_flat = xT.reshape(B * N, C_IN + 3)
    sT = jnp.transpose(support_xyz, (0, 2, 1))

    hT = pl.pallas_call(
        _mlp_body,
        grid=(B * N // TN_MLP,),
        in_specs=[
            pl.BlockSpec((TN_MLP, C_IN + 3), lambda i: (i, 0)),
            pl.BlockSpec((C_IN + 3, C_MID), lambda i: (0, 0)),
            pl.BlockSpec((1, C_MID), lambda i: (0, 0)),
            pl.BlockSpec((C_MID, C_OUT), lambda i: (0, 0)),
            pl.BlockSpec((1, C_OUT), lambda i: (0, 0)),
        ],
        out_specs=pl.BlockSpec((TN_MLP, C_OUT), lambda i: (i, 0)),
        out_shape=jax.ShapeDtypeStruct((B * N, C_OUT), jnp.float32),
    )(x_flat, W0.T, b0[None, :], W1.T, b1[None, :])

    idx = pl.pallas_call(
        _knn_body,
        grid=(B, M // TM),
        in_specs=[
            pl.BlockSpec((1, TM, 3), lambda b, mt: (b, mt, 0)),
            pl.BlockSpec((1, 3, N), lambda b, mt: (b, 0, 0)),
        ],
        out_specs=pl.BlockSpec((1, TM, K), lambda b, mt: (b, mt, 0)),
        out_shape=jax.ShapeDtypeStruct((B, M, K), jnp.int32),
    )(query_xyz, sT)

    out_flat = _make_agg()(hT, idx.reshape(NQ * K))
    return jnp.transpose(out_flat.reshape(B, M, C_OUT), (0, 2, 1))
